# Initial kernel scaffold; baseline (speedup 1.0000x reference)
import jax
import jax.numpy as jnp
from jax.experimental import pallas as pl


def kernel(z, q, ch_ids, W1_0, b1_0, W2_0, b2_0, Wh_0, bh_0, W1_1, b1_1, W2_1, b2_1, Wh_1, bh_1, W1_2, b1_2, W2_2, b2_2, Wh_2, bh_2):
    raise NotImplementedError("write your pallas kernel here")


# speedup vs baseline: 2.6126x; 2.6126x over previous
import math
import numpy as np
import jax
import jax.numpy as jnp
from jax.experimental import pallas as pl
from jax.experimental.pallas import tpu as pltpu

_B = 4096
_K = 8192
_NCH = 8
_BLK = 256
_NBLK = _B // _BLK
_JCH = 512
_UQ = 0.9
_LQ = 0.1

_INTERPRET = False


def _build_k_tables():
    size = 33 * 128
    kup = np.zeros((size,), np.int32)
    klo = np.zeros((size,), np.int32)
    for n in range(0, _B + 1):
        nf = float(max(n, 1))
        ua = math.ceil((nf + 1.0) * _UQ) / nf
        if ua > 1.0:
            ua = _UQ
        la = math.floor((nf + 1.0) * _LQ) / nf
        if la < 0.0:
            la = _LQ
        kup[n] = min(max(int(math.floor(ua * (nf - 1.0))), 0), _B - 1)
        klo[n] = min(max(int(math.floor(la * (nf - 1.0))), 0), _B - 1)
    return kup.reshape(33, 128), klo.reshape(33, 128)


_KUP_TAB, _KLO_TAB = _build_k_tables()


def _k1(q_col, q_row, ch_col, ch_row, kup_tab, klo_tab, z, Wc, b1r, Whall, W2c,
        b2c, bhr, heads_out, s1_out, s2_out):
    p = pl.program_id(0)

    zb = z[...]
    hh = jnp.dot(zb.astype(jnp.bfloat16), Wc[...].astype(jnp.bfloat16),
                 preferred_element_type=jnp.float32) + b1r[...]
    hh = 0.5 * hh * (1.0 + jax.lax.erf(hh * np.float32(1.0 / math.sqrt(2.0))))
    WW = jnp.dot(Whall[...], W2c[...], preferred_element_type=jnp.float32)
    CC = jnp.dot(Whall[...], b2c[...], preferred_element_type=jnp.float32)
    cols = []
    for t in range(3):
        ht = (hh[:, 2 * t:2 * t + 1] * WW[t:t + 1, 2 * t:2 * t + 1]
              + hh[:, 2 * t + 1:2 * t + 2] * WW[t:t + 1, 2 * t + 1:2 * t + 2]
              + (CC[t:t + 1, t:t + 1] + bhr[:, t:t + 1]))
        cols.append(ht)
    cols.append(jnp.zeros((_BLK, 5), jnp.float32))
    heads_out[...] = jnp.concatenate(cols, axis=1)

    qi = q_col[...]
    ci = ch_col[...]
    qj = q_row[...]
    cj = ch_row[...]
    igl = p * _BLK + jax.lax.broadcasted_iota(jnp.int32, (_BLK, 1), 0)

    c_lt = jnp.zeros((_BLK, 1), jnp.int32)
    c_le = jnp.zeros((_BLK, 1), jnp.int32)
    rnk = jnp.zeros((_BLK, 1), jnp.int32)
    for jc in range(_B // _JCH):
        sl = slice(jc * _JCH, (jc + 1) * _JCH)
        qjc = qj[:, sl]
        cjc = cj[:, sl]
        same = cjc == ci
        jdx = jc * _JCH + jax.lax.broadcasted_iota(jnp.int32, (_BLK, _JCH), 1)
        lt = same & (qjc < qi)
        le = same & (qjc <= qi)
        bf = same & (jdx < igl)
        c_lt = c_lt + jnp.sum(lt.astype(jnp.float32), axis=1, keepdims=True).astype(jnp.int32)
        c_le = c_le + jnp.sum(le.astype(jnp.float32), axis=1, keepdims=True).astype(jnp.int32)
        rnk = rnk + jnp.sum(bf.astype(jnp.float32), axis=1, keepdims=True).astype(jnp.int32)

    tab_idx = (jax.lax.broadcasted_iota(jnp.int32, (33, 128), 0) * 128
               + jax.lax.broadcasted_iota(jnp.int32, (33, 128), 1))
    kup_i = jnp.zeros((_BLK, 1), jnp.int32)
    klo_i = jnp.zeros((_BLK, 1), jnp.int32)
    for c in range(_NCH):
        n_c = jnp.sum((cj == c).astype(jnp.float32)).astype(jnp.int32)
        kup_c = jnp.sum(jnp.where(tab_idx == n_c, kup_tab[...], 0).astype(jnp.float32)).astype(jnp.int32)
        klo_c = jnp.sum(jnp.where(tab_idx == n_c, klo_tab[...], 0).astype(jnp.float32)).astype(jnp.int32)
        kup_i = jnp.where(ci == c, kup_c, kup_i)
        klo_i = jnp.where(ci == c, klo_c, klo_i)

    u = c_le >= kup_i + 1
    l = c_lt <= klo_i
    nm = (~u) & (~l)
    ui = u.astype(jnp.int32)
    li = l.astype(jnp.int32)
    ni = nm.astype(jnp.int32)
    m = ui + li + ni
    v1 = jnp.left_shift(m, 2 * ci)
    v2 = ui + jnp.left_shift(ni, jnp.int32(4)) + jnp.left_shift(li, jnp.int32(8))

    @pl.when(p == 0)
    def _():
        s1_out[...] = jnp.zeros((1, _B), jnp.int32)
        s2_out[...] = jnp.zeros((1, _B), jnp.int32)

    for jc in range(_B // _JCH):
        sl = slice(jc * _JCH, (jc + 1) * _JCH)
        jdx = jc * _JCH + jax.lax.broadcasted_iota(jnp.int32, (_BLK, _JCH), 1)
        onehot = jdx == rnk
        s1_out[:, sl] += jnp.sum(jnp.where(onehot, v1, 0).astype(jnp.float32), axis=0, keepdims=True).astype(jnp.int32)
        s2_out[:, sl] += jnp.sum(jnp.where(onehot, v2, 0).astype(jnp.float32), axis=0, keepdims=True).astype(jnp.int32)


def _k2(q2, ch2, hu, hn, hl, s1, s2, ref_out, qwa_out):
    S1 = s1[...]
    S2 = s2[...]
    is_u = (S2 & 15) > 0
    is_n = (jax.lax.shift_right_logical(S2, jnp.int32(4)) & 15) > 0
    is_l = (jax.lax.shift_right_logical(S2, jnp.int32(8)) & 15) > 0
    logits = jnp.where(is_u, hu[...], jnp.zeros_like(hu[...]))
    logits = jnp.where(is_n, hn[...], logits)
    logits = jnp.where(is_l, hl[...], logits)

    idx = (jax.lax.broadcasted_iota(jnp.int32, (32, 128), 0) * 128
           + jax.lax.broadcasted_iota(jnp.int32, (32, 128), 1))
    ch = ch2[...]
    qwa = jnp.zeros((32, 128), jnp.float32)
    for c in range(_NCH):
        n_c = jnp.sum((ch == c).astype(jnp.float32)).astype(jnp.int32)
        valid = idx < n_c
        xm = jnp.max(jnp.where(valid, logits, -jnp.inf))
        e = jnp.exp(logits - xm)
        mfield = (jax.lax.shift_right_logical(S1, jnp.int32(2 * c)) & 3).astype(jnp.float32)
        denom = jnp.sum(jnp.where(valid, e * mfield, 0.0))
        qwa = jnp.where(valid, e / denom, qwa)
    qwa_out[...] = qwa
    ref_out[...] = q2[...] * qwa


def kernel(z, q, ch_ids, W1_0, b1_0, W2_0, b2_0, Wh_0, bh_0,
           W1_1, b1_1, W2_1, b2_1, Wh_1, bh_1,
           W1_2, b1_2, W2_2, b2_2, Wh_2, bh_2):
    with jax.enable_x64(False):
        return _impl(z, q, ch_ids, W1_0, b1_0, W2_0, b2_0, Wh_0, bh_0,
                     W1_1, b1_1, W2_1, b2_1, Wh_1, bh_1,
                     W1_2, b1_2, W2_2, b2_2, Wh_2, bh_2)


def _impl(z, q, ch_ids, W1_0, b1_0, W2_0, b2_0, Wh_0, bh_0,
          W1_1, b1_1, W2_1, b2_1, Wh_1, bh_1,
          W1_2, b1_2, W2_2, b2_2, Wh_2, bh_2):
    q = q.astype(jnp.float32)
    ch = ch_ids.astype(jnp.int32)
    z2 = z.reshape(_B, _K)
    q_col = q.reshape(_B, 1)
    q_row = q.reshape(1, _B)
    ch_col = ch.reshape(_B, 1)
    ch_row = ch.reshape(1, _B)
    Wc = jnp.concatenate([W1_0.T, W1_1.T, W1_2.T,
                          jnp.zeros((_K, 2), jnp.float32)], axis=1)
    b1r = jnp.concatenate([b1_0, b1_1, b1_2,
                           jnp.zeros((2,), jnp.float32)]).reshape(1, 8)
    Whall = jnp.concatenate([Wh_0, Wh_1, Wh_2,
                             jnp.zeros((5, _K), jnp.float32)], axis=0)
    W2c = jnp.concatenate([W2_0, W2_1, W2_2,
                           jnp.zeros((_K, 2), jnp.float32)], axis=1)
    b2c = jnp.stack([b2_0, b2_1, b2_2] + [jnp.zeros((_K,), jnp.float32)] * 5,
                    axis=1)
    bhr = jnp.concatenate([bh_0, bh_1, bh_2,
                           jnp.zeros((5,), jnp.float32)]).reshape(1, 8)
    kup_tab = jnp.asarray(_KUP_TAB)
    klo_tab = jnp.asarray(_KLO_TAB)

    heads, s1, s2 = pl.pallas_call(
        _k1,
        grid=(_NBLK,),
        in_specs=[
            pl.BlockSpec((_BLK, 1), lambda i: (i, 0)),
            pl.BlockSpec((1, _B), lambda i: (0, 0)),
            pl.BlockSpec((_BLK, 1), lambda i: (i, 0)),
            pl.BlockSpec((1, _B), lambda i: (0, 0)),
            pl.BlockSpec((33, 128), lambda i: (0, 0)),
            pl.BlockSpec((33, 128), lambda i: (0, 0)),
            pl.BlockSpec((_BLK, _K), lambda i: (i, 0)),
            pl.BlockSpec((_K, 8), lambda i: (0, 0)),
            pl.BlockSpec((1, 8), lambda i: (0, 0)),
            pl.BlockSpec((8, _K), lambda i: (0, 0)),
            pl.BlockSpec((_K, 8), lambda i: (0, 0)),
            pl.BlockSpec((_K, 8), lambda i: (0, 0)),
            pl.BlockSpec((1, 8), lambda i: (0, 0)),
        ],
        out_specs=[
            pl.BlockSpec((_BLK, 8), lambda i: (i, 0)),
            pl.BlockSpec((1, _B), lambda i: (0, 0)),
            pl.BlockSpec((1, _B), lambda i: (0, 0)),
        ],
        out_shape=[
            jax.ShapeDtypeStruct((_B, 8), jnp.float32),
            jax.ShapeDtypeStruct((1, _B), jnp.int32),
            jax.ShapeDtypeStruct((1, _B), jnp.int32),
        ],
        interpret=_INTERPRET,
    )(q_col, q_row, ch_col, ch_row, kup_tab, klo_tab, z2, Wc, b1r, Whall, W2c,
      b2c, bhr)

    q2 = q.reshape(32, 128)
    ch2 = ch.reshape(32, 128)
    hu = heads[:, 0].reshape(32, 128)
    hn = heads[:, 1].reshape(32, 128)
    hl = heads[:, 2].reshape(32, 128)
    s1_2 = s1.reshape(32, 128)
    s2_2 = s2.reshape(32, 128)

    refined, qwa = pl.pallas_call(
        _k2,
        out_shape=[
            jax.ShapeDtypeStruct((32, 128), jnp.float32),
            jax.ShapeDtypeStruct((32, 128), jnp.float32),
        ],
        interpret=_INTERPRET,
    )(q2, ch2, hu, hn, hl, s1_2, s2_2)

    return refined.reshape(_B), qwa.reshape(_B)


# speedup vs baseline: 2.8730x; 1.0997x over previous
import math
import numpy as np
import jax
import jax.numpy as jnp
from jax.experimental import pallas as pl
from jax.experimental.pallas import tpu as pltpu

_B = 4096
_K = 8192
_NCH = 8
_BLK = 256
_NBLK = _B // _BLK
_JCH = 512
_UQ = 0.9
_LQ = 0.1

_INTERPRET = False


def _build_k_tables():
    size = 33 * 128
    kup = np.zeros((size,), np.int32)
    klo = np.zeros((size,), np.int32)
    for n in range(0, _B + 1):
        nf = float(max(n, 1))
        ua = math.ceil((nf + 1.0) * _UQ) / nf
        if ua > 1.0:
            ua = _UQ
        la = math.floor((nf + 1.0) * _LQ) / nf
        if la < 0.0:
            la = _LQ
        kup[n] = min(max(int(math.floor(ua * (nf - 1.0))), 0), _B - 1)
        klo[n] = min(max(int(math.floor(la * (nf - 1.0))), 0), _B - 1)
    return kup.reshape(33, 128), klo.reshape(33, 128)


_KUP_TAB, _KLO_TAB = _build_k_tables()


def _k1(q_col, q_row, ch_col, ch_row, kup_tab, klo_tab, z, Wc, b1r, Whall, W2c,
        b2c, bhr, heads_out, s1_out, s2_out, thr_scr, base_scr):
    p = pl.program_id(0)

    zb = z[...]
    hh = jnp.dot(zb.astype(jnp.bfloat16), Wc[...].astype(jnp.bfloat16),
                 preferred_element_type=jnp.float32) + b1r[...]
    hh = 0.5 * hh * (1.0 + jax.lax.erf(hh * np.float32(1.0 / math.sqrt(2.0))))
    WW = jnp.dot(Whall[...], W2c[...], preferred_element_type=jnp.float32)
    CC = jnp.dot(Whall[...], b2c[...], preferred_element_type=jnp.float32)
    cols = []
    for t in range(3):
        ht = (hh[:, 2 * t:2 * t + 1] * WW[t:t + 1, 2 * t:2 * t + 1]
              + hh[:, 2 * t + 1:2 * t + 2] * WW[t:t + 1, 2 * t + 1:2 * t + 2]
              + (CC[t:t + 1, t:t + 1] + bhr[:, t:t + 1]))
        cols.append(ht)
    cols.append(jnp.zeros((_BLK, 5), jnp.float32))
    heads_out[...] = jnp.concatenate(cols, axis=1)

    qj = q_row[...]
    cj = ch_row[...]
    qb_row = jax.lax.bitcast_convert_type(qj, jnp.int32)

    @pl.when(p == 0)
    def _():
        tab_idx = (jax.lax.broadcasted_iota(jnp.int32, (33, 128), 0) * 128
                   + jax.lax.broadcasted_iota(jnp.int32, (33, 128), 1))
        rows = jax.lax.broadcasted_iota(jnp.int32, (16, 1), 0)
        chv = rows & 7
        chmask = cj == chv
        kvec = jnp.zeros((16, 1), jnp.int32)
        for c in range(_NCH):
            n_c = jnp.sum((cj == c).astype(jnp.float32)).astype(jnp.int32)
            kup_c = jnp.sum(jnp.where(tab_idx == n_c, kup_tab[...], 0).astype(jnp.float32)).astype(jnp.int32)
            klo_c = jnp.sum(jnp.where(tab_idx == n_c, klo_tab[...], 0).astype(jnp.float32)).astype(jnp.int32)
            kvec = jnp.where(rows == c, kup_c, kvec)
            kvec = jnp.where(rows == c + _NCH, klo_c, kvec)

        def body(_, carry):
            lo, hi = carry
            mid = lo + jnp.right_shift(hi - lo, 1)
            cnt = jnp.sum((chmask & (qb_row <= mid)).astype(jnp.float32),
                          axis=1, keepdims=True).astype(jnp.int32)
            ge = cnt >= kvec + 1
            return (jnp.where(ge, lo, mid + 1), jnp.where(ge, mid, hi))

        lo0 = jnp.zeros((16, 1), jnp.int32)
        hi0 = jnp.full((16, 1), 1 << 30, jnp.int32)
        lo_f, _hi = jax.lax.fori_loop(0, 31, body, (lo0, hi0))
        thr_scr[...] = lo_f
        base_scr[...] = jnp.zeros((1, _NCH), jnp.int32)
        s1_out[...] = jnp.zeros((1, _B), jnp.float32)
        s2_out[...] = jnp.zeros((1, _B), jnp.float32)

    qi = q_col[...]
    ci = ch_col[...]
    qbi = jax.lax.bitcast_convert_type(qi, jnp.int32)
    upb = jnp.zeros((_BLK, 1), jnp.int32)
    lob = jnp.zeros((_BLK, 1), jnp.int32)
    for c in range(_NCH):
        upb = jnp.where(ci == c, thr_scr[c:c + 1, 0:1], upb)
        lob = jnp.where(ci == c, thr_scr[c + _NCH:c + _NCH + 1, 0:1], lob)
    u = qbi >= upb
    l = qbi <= lob
    nm = (~u) & (~l)
    ui = u.astype(jnp.int32)
    li = l.astype(jnp.int32)
    ni = nm.astype(jnp.int32)
    m = ui + li + ni
    v1f = jnp.left_shift(m, 2 * ci).astype(jnp.float32)
    v2f = (ui + jnp.left_shift(ni, 4) + jnp.left_shift(li, 8)).astype(jnp.float32)

    chb = ch_row[0:1, pl.ds(p * _BLK, _BLK)]
    same_i = chb == ci
    jloc = jax.lax.broadcasted_iota(jnp.int32, (_BLK, _BLK), 1)
    iloc = jax.lax.broadcasted_iota(jnp.int32, (_BLK, _BLK), 0)
    intra = jnp.sum((same_i & (jloc < iloc)).astype(jnp.float32),
                    axis=1, keepdims=True).astype(jnp.int32)
    base = base_scr[...]
    baser = jnp.zeros((_BLK, 1), jnp.int32)
    cnts = []
    for c in range(_NCH):
        baser = jnp.where(ci == c, base[0:1, c:c + 1], baser)
        cnts.append(jnp.sum((ci == c).astype(jnp.float32), keepdims=True).astype(jnp.int32))
    base_scr[...] = base + jnp.concatenate(cnts, axis=1)
    rnk = baser + intra

    for jc in range(_B // _JCH):
        sl = slice(jc * _JCH, (jc + 1) * _JCH)
        jdx = jc * _JCH + jax.lax.broadcasted_iota(jnp.int32, (_BLK, _JCH), 1)
        onehot = jdx == rnk
        s1_out[:, sl] += jnp.sum(jnp.where(onehot, v1f, 0.0), axis=0, keepdims=True)
        s2_out[:, sl] += jnp.sum(jnp.where(onehot, v2f, 0.0), axis=0, keepdims=True)


def _k2(q2, ch2, hu, hn, hl, s1, s2, ref_out, qwa_out):
    S1 = s1[...].astype(jnp.int32)
    SS = s2[...].astype(jnp.int32)
    is_u = (SS & 15) > 0
    is_n = (jax.lax.shift_right_logical(SS, jnp.int32(4)) & 15) > 0
    is_l = (jax.lax.shift_right_logical(SS, jnp.int32(8)) & 15) > 0
    logits = jnp.where(is_u, hu[...], jnp.zeros_like(hu[...]))
    logits = jnp.where(is_n, hn[...], logits)
    logits = jnp.where(is_l, hl[...], logits)

    idx = (jax.lax.broadcasted_iota(jnp.int32, (32, 128), 0) * 128
           + jax.lax.broadcasted_iota(jnp.int32, (32, 128), 1))
    ch = ch2[...]
    qwa = jnp.zeros((32, 128), jnp.float32)
    for c in range(_NCH):
        n_c = jnp.sum((ch == c).astype(jnp.float32)).astype(jnp.int32)
        valid = idx < n_c
        xm = jnp.max(jnp.where(valid, logits, -jnp.inf))
        e = jnp.exp(logits - xm)
        mfield = (jax.lax.shift_right_logical(S1, jnp.int32(2 * c)) & 3).astype(jnp.float32)
        denom = jnp.sum(jnp.where(valid, e * mfield, 0.0))
        qwa = jnp.where(valid, e / denom, qwa)
    qwa_out[...] = qwa
    ref_out[...] = q2[...] * qwa


def kernel(z, q, ch_ids, W1_0, b1_0, W2_0, b2_0, Wh_0, bh_0,
           W1_1, b1_1, W2_1, b2_1, Wh_1, bh_1,
           W1_2, b1_2, W2_2, b2_2, Wh_2, bh_2):
    with jax.enable_x64(False):
        return _impl(z, q, ch_ids, W1_0, b1_0, W2_0, b2_0, Wh_0, bh_0,
                     W1_1, b1_1, W2_1, b2_1, Wh_1, bh_1,
                     W1_2, b1_2, W2_2, b2_2, Wh_2, bh_2)


def _impl(z, q, ch_ids, W1_0, b1_0, W2_0, b2_0, Wh_0, bh_0,
          W1_1, b1_1, W2_1, b2_1, Wh_1, bh_1,
          W1_2, b1_2, W2_2, b2_2, Wh_2, bh_2):
    q = q.astype(jnp.float32)
    ch = ch_ids.astype(jnp.int32)
    z2 = z.reshape(_B, _K)
    q_col = q.reshape(_B, 1)
    q_row = q.reshape(1, _B)
    ch_col = ch.reshape(_B, 1)
    ch_row = ch.reshape(1, _B)
    Wc = jnp.concatenate([W1_0.T, W1_1.T, W1_2.T,
                          jnp.zeros((_K, 2), jnp.float32)], axis=1)
    b1r = jnp.concatenate([b1_0, b1_1, b1_2,
                           jnp.zeros((2,), jnp.float32)]).reshape(1, 8)
    Whall = jnp.concatenate([Wh_0, Wh_1, Wh_2,
                             jnp.zeros((5, _K), jnp.float32)], axis=0)
    W2c = jnp.concatenate([W2_0, W2_1, W2_2,
                           jnp.zeros((_K, 2), jnp.float32)], axis=1)
    b2c = jnp.stack([b2_0, b2_1, b2_2] + [jnp.zeros((_K,), jnp.float32)] * 5,
                    axis=1)
    bhr = jnp.concatenate([bh_0, bh_1, bh_2,
                           jnp.zeros((5,), jnp.float32)]).reshape(1, 8)
    kup_tab = jnp.asarray(_KUP_TAB)
    klo_tab = jnp.asarray(_KLO_TAB)

    heads, s1o, s2o = pl.pallas_call(
        _k1,
        grid=(_NBLK,),
        in_specs=[
            pl.BlockSpec((_BLK, 1), lambda i: (i, 0)),
            pl.BlockSpec((1, _B), lambda i: (0, 0)),
            pl.BlockSpec((_BLK, 1), lambda i: (i, 0)),
            pl.BlockSpec((1, _B), lambda i: (0, 0)),
            pl.BlockSpec((33, 128), lambda i: (0, 0)),
            pl.BlockSpec((33, 128), lambda i: (0, 0)),
            pl.BlockSpec((_BLK, _K), lambda i: (i, 0)),
            pl.BlockSpec((_K, 8), lambda i: (0, 0)),
            pl.BlockSpec((1, 8), lambda i: (0, 0)),
            pl.BlockSpec((8, _K), lambda i: (0, 0)),
            pl.BlockSpec((_K, 8), lambda i: (0, 0)),
            pl.BlockSpec((_K, 8), lambda i: (0, 0)),
            pl.BlockSpec((1, 8), lambda i: (0, 0)),
        ],
        out_specs=[
            pl.BlockSpec((_BLK, 8), lambda i: (i, 0)),
            pl.BlockSpec((1, _B), lambda i: (0, 0)),
            pl.BlockSpec((1, _B), lambda i: (0, 0)),
        ],
        out_shape=[
            jax.ShapeDtypeStruct((_B, 8), jnp.float32),
            jax.ShapeDtypeStruct((1, _B), jnp.float32),
            jax.ShapeDtypeStruct((1, _B), jnp.float32),
        ],
        scratch_shapes=[
            pltpu.VMEM((16, 1), jnp.int32),
            pltpu.VMEM((1, _NCH), jnp.int32),
        ],
        interpret=_INTERPRET,
    )(q_col, q_row, ch_col, ch_row, kup_tab, klo_tab, z2, Wc, b1r, Whall, W2c,
      b2c, bhr)

    q2 = q.reshape(32, 128)
    ch2 = ch.reshape(32, 128)
    hu = heads[:, 0].reshape(32, 128)
    hn = heads[:, 1].reshape(32, 128)
    hl = heads[:, 2].reshape(32, 128)
    s1d = s1o.reshape(32, 128)
    s2d = s2o.reshape(32, 128)

    refined, qwa = pl.pallas_call(
        _k2,
        out_shape=[
            jax.ShapeDtypeStruct((32, 128), jnp.float32),
            jax.ShapeDtypeStruct((32, 128), jnp.float32),
        ],
        interpret=_INTERPRET,
    )(q2, ch2, hu, hn, hl, s1d, s2d)

    return refined.reshape(_B), qwa.reshape(_B)


# speedup vs baseline: 2.8985x; 1.0088x over previous
import math
import numpy as np
import jax
import jax.numpy as jnp
from jax.experimental import pallas as pl
from jax.experimental.pallas import tpu as pltpu

_B = 4096
_K = 8192
_NCH = 8
_BLK = 256
_NBLK = _B // _BLK
_JCH = 512
_UQ = 0.9
_LQ = 0.1

_INTERPRET = False


def _build_k_tables():
    size = 33 * 128
    kup = np.zeros((size,), np.int32)
    klo = np.zeros((size,), np.int32)
    for n in range(0, _B + 1):
        nf = float(max(n, 1))
        ua = math.ceil((nf + 1.0) * _UQ) / nf
        if ua > 1.0:
            ua = _UQ
        la = math.floor((nf + 1.0) * _LQ) / nf
        if la < 0.0:
            la = _LQ
        kup[n] = min(max(int(math.floor(ua * (nf - 1.0))), 0), _B - 1)
        klo[n] = min(max(int(math.floor(la * (nf - 1.0))), 0), _B - 1)
    return kup.reshape(33, 128), klo.reshape(33, 128)


_KUP_TAB, _KLO_TAB = _build_k_tables()


def _k1(q_col, q_row, ch_col, ch_row, kup_tab, klo_tab, z, Wc, b1r, Whall, W2c,
        b2c, bhr, heads_out, s1_out, thr_scr, base_scr):
    p = pl.program_id(0)

    zb = z[...]
    hh = jnp.dot(zb.astype(jnp.bfloat16), Wc[...].astype(jnp.bfloat16),
                 preferred_element_type=jnp.float32) + b1r[...]
    hh = 0.5 * hh * (1.0 + jax.lax.erf(hh * np.float32(1.0 / math.sqrt(2.0))))
    WW = jnp.dot(Whall[...], W2c[...], preferred_element_type=jnp.float32)
    CC = jnp.dot(Whall[...], b2c[...], preferred_element_type=jnp.float32)
    cols = []
    for t in range(3):
        ht = (hh[:, 2 * t:2 * t + 1] * WW[t:t + 1, 2 * t:2 * t + 1]
              + hh[:, 2 * t + 1:2 * t + 2] * WW[t:t + 1, 2 * t + 1:2 * t + 2]
              + (CC[t:t + 1, t:t + 1] + bhr[:, t:t + 1]))
        cols.append(ht)
    cols.append(jnp.zeros((_BLK, 5), jnp.float32))
    heads_out[...] = jnp.concatenate(cols, axis=1)

    qj = q_row[...]
    cj = ch_row[...]
    qb_row = jax.lax.bitcast_convert_type(qj, jnp.int32)

    @pl.when(p == 0)
    def _():
        tab_idx = (jax.lax.broadcasted_iota(jnp.int32, (33, 128), 0) * 128
                   + jax.lax.broadcasted_iota(jnp.int32, (33, 128), 1))
        rows = jax.lax.broadcasted_iota(jnp.int32, (16, 1), 0)
        chv = rows & 7
        chmask = cj == chv
        kvec = jnp.zeros((16, 1), jnp.int32)
        for c in range(_NCH):
            n_c = jnp.sum((cj == c).astype(jnp.float32)).astype(jnp.int32)
            kup_c = jnp.sum(jnp.where(tab_idx == n_c, kup_tab[...], 0).astype(jnp.float32)).astype(jnp.int32)
            klo_c = jnp.sum(jnp.where(tab_idx == n_c, klo_tab[...], 0).astype(jnp.float32)).astype(jnp.int32)
            kvec = jnp.where(rows == c, kup_c, kvec)
            kvec = jnp.where(rows == c + _NCH, klo_c, kvec)

        def body(_, carry):
            lo, hi = carry
            mid = lo + jnp.right_shift(hi - lo, 1)
            cnt = jnp.sum((chmask & (qb_row <= mid)).astype(jnp.float32),
                          axis=1, keepdims=True).astype(jnp.int32)
            ge = cnt >= kvec + 1
            return (jnp.where(ge, lo, mid + 1), jnp.where(ge, mid, hi))

        lo0 = jnp.zeros((16, 1), jnp.int32)
        hi0 = jnp.full((16, 1), 1 << 30, jnp.int32)
        lo_f, _hi = jax.lax.fori_loop(0, 31, body, (lo0, hi0))
        thr_scr[...] = lo_f
        base_scr[...] = jnp.zeros((1, _NCH), jnp.int32)
        s1_out[...] = jnp.zeros((1, _B), jnp.float32)

    qi = q_col[...]
    ci = ch_col[...]
    qbi = jax.lax.bitcast_convert_type(qi, jnp.int32)
    upb = jnp.zeros((_BLK, 1), jnp.int32)
    lob = jnp.zeros((_BLK, 1), jnp.int32)
    for c in range(_NCH):
        upb = jnp.where(ci == c, thr_scr[c:c + 1, 0:1], upb)
        lob = jnp.where(ci == c, thr_scr[c + _NCH:c + _NCH + 1, 0:1], lob)
    u = qbi >= upb
    l = qbi <= lob
    ui = u.astype(jnp.int32)
    li = l.astype(jnp.int32)
    vf = jnp.left_shift(ui + 2 * li, 2 * ci).astype(jnp.float32)

    chb = ch_row[0:1, pl.ds(p * _BLK, _BLK)]
    same_i = chb == ci
    jloc = jax.lax.broadcasted_iota(jnp.int32, (_BLK, _BLK), 1)
    iloc = jax.lax.broadcasted_iota(jnp.int32, (_BLK, _BLK), 0)
    intra = jnp.sum((same_i & (jloc < iloc)).astype(jnp.float32),
                    axis=1, keepdims=True).astype(jnp.int32)
    base = base_scr[...]
    baser = jnp.zeros((_BLK, 1), jnp.int32)
    cnts = []
    for c in range(_NCH):
        baser = jnp.where(ci == c, base[0:1, c:c + 1], baser)
        cnts.append(jnp.sum((ci == c).astype(jnp.float32), keepdims=True).astype(jnp.int32))
    base_scr[...] = base + jnp.concatenate(cnts, axis=1)
    rnk = baser + intra

    for jc in range(_B // _JCH):
        @pl.when(p >= 2 * jc)
        def _(jc=jc):
            sl = slice(jc * _JCH, (jc + 1) * _JCH)
            jdx = jc * _JCH + jax.lax.broadcasted_iota(jnp.int32, (_BLK, _JCH), 1)
            onehot = jdx == rnk
            s1_out[:, sl] += jnp.sum(jnp.where(onehot, vf, 0.0), axis=0,
                                     keepdims=True)


def _k2(q2, ch2, hu, hn, hl, s1, ref_out, qwa_out):
    S = s1[...].astype(jnp.int32)
    idx = (jax.lax.broadcasted_iota(jnp.int32, (32, 128), 0) * 128
           + jax.lax.broadcasted_iota(jnp.int32, (32, 128), 1))
    ch = ch2[...]
    ns, valids, mfs = [], [], []
    acc_u = jnp.zeros((32, 128), jnp.int32)
    acc_n = jnp.zeros((32, 128), jnp.int32)
    acc_l = jnp.zeros((32, 128), jnp.int32)
    for c in range(_NCH):
        n_c = jnp.sum((ch == c).astype(jnp.float32)).astype(jnp.int32)
        valid = idx < n_c
        u_c = jax.lax.shift_right_logical(S, jnp.int32(2 * c)) & 1
        l_c = jax.lax.shift_right_logical(S, jnp.int32(2 * c + 1)) & 1
        vi = valid.astype(jnp.int32)
        n_c_flag = vi * (1 - u_c) * (1 - l_c)
        acc_u = acc_u + u_c
        acc_n = acc_n + n_c_flag
        acc_l = acc_l + l_c
        valids.append(valid)
        mfs.append((u_c + l_c + n_c_flag).astype(jnp.float32))
    logits = jnp.where(acc_u > 0, hu[...], jnp.zeros_like(hu[...]))
    logits = jnp.where(acc_n > 0, hn[...], logits)
    logits = jnp.where(acc_l > 0, hl[...], logits)
    qwa = jnp.zeros((32, 128), jnp.float32)
    for c in range(_NCH):
        valid = valids[c]
        xm = jnp.max(jnp.where(valid, logits, -jnp.inf))
        e = jnp.exp(logits - xm)
        denom = jnp.sum(jnp.where(valid, e * mfs[c], 0.0))
        qwa = jnp.where(valid, e / denom, qwa)
    qwa_out[...] = qwa
    ref_out[...] = q2[...] * qwa


def kernel(z, q, ch_ids, W1_0, b1_0, W2_0, b2_0, Wh_0, bh_0,
           W1_1, b1_1, W2_1, b2_1, Wh_1, bh_1,
           W1_2, b1_2, W2_2, b2_2, Wh_2, bh_2):
    with jax.enable_x64(False):
        return _impl(z, q, ch_ids, W1_0, b1_0, W2_0, b2_0, Wh_0, bh_0,
                     W1_1, b1_1, W2_1, b2_1, Wh_1, bh_1,
                     W1_2, b1_2, W2_2, b2_2, Wh_2, bh_2)


def _impl(z, q, ch_ids, W1_0, b1_0, W2_0, b2_0, Wh_0, bh_0,
          W1_1, b1_1, W2_1, b2_1, Wh_1, bh_1,
          W1_2, b1_2, W2_2, b2_2, Wh_2, bh_2):
    q = q.astype(jnp.float32)
    ch = ch_ids.astype(jnp.int32)
    z2 = z.reshape(_B, _K)
    q_col = q.reshape(_B, 1)
    q_row = q.reshape(1, _B)
    ch_col = ch.reshape(_B, 1)
    ch_row = ch.reshape(1, _B)
    Wc = jnp.concatenate([W1_0.T, W1_1.T, W1_2.T,
                          jnp.zeros((_K, 2), jnp.float32)], axis=1)
    b1r = jnp.concatenate([b1_0, b1_1, b1_2,
                           jnp.zeros((2,), jnp.float32)]).reshape(1, 8)
    Whall = jnp.concatenate([Wh_0, Wh_1, Wh_2,
                             jnp.zeros((5, _K), jnp.float32)], axis=0)
    W2c = jnp.concatenate([W2_0, W2_1, W2_2,
                           jnp.zeros((_K, 2), jnp.float32)], axis=1)
    b2c = jnp.stack([b2_0, b2_1, b2_2] + [jnp.zeros((_K,), jnp.float32)] * 5,
                    axis=1)
    bhr = jnp.concatenate([bh_0, bh_1, bh_2,
                           jnp.zeros((5,), jnp.float32)]).reshape(1, 8)
    kup_tab = jnp.asarray(_KUP_TAB)
    klo_tab = jnp.asarray(_KLO_TAB)

    heads, s1o = pl.pallas_call(
        _k1,
        grid=(_NBLK,),
        in_specs=[
            pl.BlockSpec((_BLK, 1), lambda i: (i, 0)),
            pl.BlockSpec((1, _B), lambda i: (0, 0)),
            pl.BlockSpec((_BLK, 1), lambda i: (i, 0)),
            pl.BlockSpec((1, _B), lambda i: (0, 0)),
            pl.BlockSpec((33, 128), lambda i: (0, 0)),
            pl.BlockSpec((33, 128), lambda i: (0, 0)),
            pl.BlockSpec((_BLK, _K), lambda i: (i, 0)),
            pl.BlockSpec((_K, 8), lambda i: (0, 0)),
            pl.BlockSpec((1, 8), lambda i: (0, 0)),
            pl.BlockSpec((8, _K), lambda i: (0, 0)),
            pl.BlockSpec((_K, 8), lambda i: (0, 0)),
            pl.BlockSpec((_K, 8), lambda i: (0, 0)),
            pl.BlockSpec((1, 8), lambda i: (0, 0)),
        ],
        out_specs=[
            pl.BlockSpec((_BLK, 8), lambda i: (i, 0)),
            pl.BlockSpec((1, _B), lambda i: (0, 0)),
        ],
        out_shape=[
            jax.ShapeDtypeStruct((_B, 8), jnp.float32),
            jax.ShapeDtypeStruct((1, _B), jnp.float32),
        ],
        scratch_shapes=[
            pltpu.VMEM((16, 1), jnp.int32),
            pltpu.VMEM((1, _NCH), jnp.int32),
        ],
        interpret=_INTERPRET,
    )(q_col, q_row, ch_col, ch_row, kup_tab, klo_tab, z2, Wc, b1r, Whall, W2c,
      b2c, bhr)

    q2 = q.reshape(32, 128)
    ch2 = ch.reshape(32, 128)
    hu = heads[:, 0].reshape(32, 128)
    hn = heads[:, 1].reshape(32, 128)
    hl = heads[:, 2].reshape(32, 128)
    s1d = s1o.reshape(32, 128)

    refined, qwa = pl.pallas_call(
        _k2,
        out_shape=[
            jax.ShapeDtypeStruct((32, 128), jnp.float32),
            jax.ShapeDtypeStruct((32, 128), jnp.float32),
        ],
        interpret=_INTERPRET,
    )(q2, ch2, hu, hn, hl, s1d)

    return refined.reshape(_B), qwa.reshape(_B)
